# baseline (speedup 1.0000x reference)
import functools

import jax
import jax.numpy as jnp
from jax import lax
from jax.experimental import pallas as pl
from jax.experimental.pallas import tpu as pltpu
from jax.experimental.pallas import tpu_sc as plsc

B, S, D, E, K = 2, 4096, 2048, 64, 2
TOK_BLK = 1024
N = B * S

_info = plsc.get_sparse_core_info()
NC, NS, L = _info.num_cores, _info.num_subcores, _info.num_lanes
NW = NC * NS
TOK_PER_W = N // NW
GROUPS = TOK_PER_W // L


def _proj_kernel(x_ref, wt_ref, b_ref, h_ref):
    h_ref[...] = (
        jnp.dot(x_ref[...], wt_ref[...], preferred_element_type=jnp.float32)
        + b_ref[...]
    )


def _splat_last(vec):
    idx = jnp.full((L,), L - 1, jnp.int32)
    return lax.gather(
        vec,
        idx[:, None],
        dimension_numbers=lax.GatherDimensionNumbers(
            offset_dims=(), collapsed_slice_dims=(0,), start_index_map=(0,)
        ),
        slice_sizes=(1,),
        mode=lax.GatherScatterMode.PROMISE_IN_BOUNDS,
    )


def _vmax_splat(vec):
    return _splat_last(plsc.cummax(vec))


def _vmin_splat(vec):
    return -_splat_last(plsc.cummax(-vec))


def _route_body(h_hbm, out_hbm, h_v, w_v):
    c = lax.axis_index("c")
    s = lax.axis_index("s")
    wid = s * NC + c
    base = wid * (TOK_PER_W * E)
    pltpu.sync_copy(h_hbm.at[pl.ds(base, TOK_PER_W * E)], h_v)
    lane = lax.iota(jnp.int32, L)
    neg_inf = jnp.full((L,), -jnp.inf, jnp.float32)
    one = jnp.full((L,), 1.0, jnp.float32)
    zero = jnp.zeros((L,), jnp.float32)
    e_ids = [jnp.full((L,), float(e), jnp.float32) for e in range(E)]
    big = jnp.full((L,), float(E), jnp.float32)
    e_off = [jnp.full((L,), e, jnp.int32) for e in range(E)]
    ev = jnp.full((L,), E, jnp.int32)

    def group(g, carry):
        flat0 = (g * L + lane) * ev
        idx = [flat0 + e_off[e] for e in range(E)]
        hs = [plsc.load_gather(h_v, [idx[e]]) for e in range(E)]
        m1 = hs[0]
        for e in range(1, E):
            m1 = jnp.maximum(m1, hs[e])
        t1 = [jnp.where(hs[e] == m1, e_ids[e], big) for e in range(E)]
        i1 = t1[0]
        for e in range(1, E):
            i1 = jnp.minimum(i1, t1[e])
        sel1 = [t1[e] == i1 for e in range(E)]
        h2 = [jnp.where(sel1[e], neg_inf, hs[e]) for e in range(E)]
        m2 = h2[0]
        for e in range(1, E):
            m2 = jnp.maximum(m2, h2[e])
        t2 = [jnp.where(h2[e] == m2, e_ids[e], big) for e in range(E)]
        i2 = t2[0]
        for e in range(1, E):
            i2 = jnp.minimum(i2, t2[e])
        e2 = jnp.exp(m2 - m1)
        z = one + e2
        w1 = one / z
        w2 = e2 / z
        for e in range(E):
            w = jnp.where(sel1[e], w1, jnp.where(t2[e] == i2, w2, zero))
            plsc.store_scatter(w_v, [idx[e]], w)
        return carry

    lax.fori_loop(0, GROUPS, group, 0)
    pltpu.sync_copy(w_v, out_hbm.at[pl.ds(base, TOK_PER_W * E)])


def _route(h_flat):
    mesh = plsc.VectorSubcoreMesh(core_axis_name="c", subcore_axis_name="s")
    return pl.kernel(
        _route_body,
        mesh=mesh,
        out_type=jax.ShapeDtypeStruct((N * E,), jnp.float32),
        compiler_params=pltpu.CompilerParams(needs_layout_passes=False),
        scratch_types=[
            pltpu.VMEM((TOK_PER_W * E,), jnp.float32),
            pltpu.VMEM((TOK_PER_W * E,), jnp.float32),
        ],
    )(h_flat)


@functools.partial(jax.jit, static_argnames=())
def kernel(x, W, b):
    xt = x.reshape(N, D)
    wt = W.T
    b2 = b.reshape(1, E)
    grid = N // TOK_BLK
    h = pl.pallas_call(
        _proj_kernel,
        grid=(grid,),
        in_specs=[
            pl.BlockSpec((TOK_BLK, D), lambda i: (i, 0)),
            pl.BlockSpec((D, E), lambda i: (0, 0)),
            pl.BlockSpec((1, E), lambda i: (0, 0)),
        ],
        out_specs=pl.BlockSpec((TOK_BLK, E), lambda i: (i, 0)),
        out_shape=jax.ShapeDtypeStruct((N, E), jnp.float32),
        compiler_params=pltpu.CompilerParams(
            dimension_semantics=("parallel",),
        ),
    )(xt, wt, b2)
    out = _route(h.reshape(N * E))
    return out.reshape(B, S, E)


# speedup vs baseline: 2.3359x; 2.3359x over previous
import functools

import jax
import jax.numpy as jnp
from jax.experimental import pallas as pl
from jax.experimental.pallas import tpu as pltpu

B, S, D, E, K = 2, 4096, 2048, 64, 2
TOK_BLK = 1024


def _router_kernel(x_ref, wt_ref, b_ref, out_ref):
    h = jnp.dot(x_ref[...], wt_ref[...], preferred_element_type=jnp.float32)
    h = h + b_ref[...]
    neg_inf = jnp.float32(-jnp.inf)
    lane = jax.lax.broadcasted_iota(jnp.int32, h.shape, 1).astype(jnp.float32)
    m1 = jnp.max(h, axis=1, keepdims=True)
    t1 = jnp.where(h == m1, lane, jnp.float32(E))
    i1 = jnp.min(t1, axis=1, keepdims=True)
    sel1 = t1 == i1
    h2 = jnp.where(sel1, neg_inf, h)
    m2 = jnp.max(h2, axis=1, keepdims=True)
    t2 = jnp.where(h2 == m2, lane, jnp.float32(E))
    i2 = jnp.min(t2, axis=1, keepdims=True)
    sel2 = t2 == i2
    e2 = jnp.exp(m2 - m1)
    z = 1.0 + e2
    out_ref[...] = jnp.where(sel1, 1.0 / z, jnp.where(sel2, e2 / z, 0.0))


@functools.partial(jax.jit, static_argnames=())
def kernel(x, W, b):
    xt = x.reshape(B * S, D)
    wt = W.T
    b2 = b.reshape(1, E)
    grid = (B * S) // TOK_BLK
    out = pl.pallas_call(
        _router_kernel,
        grid=(grid,),
        in_specs=[
            pl.BlockSpec((TOK_BLK, D), lambda i: (i, 0)),
            pl.BlockSpec((D, E), lambda i: (0, 0)),
            pl.BlockSpec((1, E), lambda i: (0, 0)),
        ],
        out_specs=pl.BlockSpec((TOK_BLK, E), lambda i: (i, 0)),
        out_shape=jax.ShapeDtypeStruct((B * S, E), jnp.float32),
        compiler_params=pltpu.CompilerParams(
            dimension_semantics=("parallel",),
        ),
    )(xt, wt, b2)
    return out.reshape(B, S, E)
